# Initial kernel scaffold; baseline (speedup 1.0000x reference)
import jax
import jax.numpy as jnp
from jax.experimental import pallas as pl


def kernel(feature_GP, feature_Node, feature_tan, feature_T, params, edge_index):
    raise NotImplementedError("write your pallas kernel here")


# speedup vs baseline: 3.9872x; 3.9872x over previous
import functools

import jax
import jax.numpy as jnp
from jax import lax
from jax.experimental import pallas as pl
from jax.experimental.pallas import tpu as pltpu
from jax.experimental.pallas import tpu_sc as plsc

F32 = jnp.float32
BF16 = jnp.bfloat16

N_NODES = 10000
N_EDGES = 320000
D = 128
TILES = 16
WORKERS = 32
NPAD = 10240
ROWS_PT = NPAD // TILES
CHUNK = 80

EPT = N_EDGES // TILES
NCH = EPT // CHUNK
EPW = N_EDGES // WORKERS
NCHW = EPW // CHUNK

EBLK = 4000
NBLK = 1000

_sc_cache = {}


def _sc_gather1_body(X_hbm, src_hbm, dst_hbm, msg_hbm,
                     src_v, dst_v, rs_v, rd_v, sem1, sem2):
    c = lax.axis_index("c")
    s = lax.axis_index("s")
    w = c * TILES + s

    def _chunk(k, _):
        eb = w * EPW + k * CHUNK
        pltpu.sync_copy(src_hbm.at[pl.ds(eb, CHUNK)], src_v)
        pltpu.sync_copy(dst_hbm.at[pl.ds(eb, CHUNK)], dst_v)
        g1 = pltpu.async_copy(X_hbm.at[src_v], rs_v, sem1)
        g2 = pltpu.async_copy(X_hbm.at[dst_v], rd_v, sem2)
        g1.wait()
        g2.wait()

        def _add(i, _):
            for j in range(D // 16):
                sl = pl.ds(j * 16, 16)
                rs_v[i, sl] = rs_v[i, sl] + rd_v[i, sl]
            return 0
        lax.fori_loop(0, CHUNK, _add, 0)

        pltpu.sync_copy(rs_v, msg_hbm.at[pl.ds(eb, CHUNK)])
        return 0

    lax.fori_loop(0, NCHW, _chunk, 0)


def _sc_gather2_body(H_hbm, src_hbm, dst_hbm, msg_hbm,
                     src_v, dst_v, rs_v, rd_v, sem1, sem2):
    c = lax.axis_index("c")
    s = lax.axis_index("s")

    def _chunk(k, _):
        eb = s * EPT + k * CHUNK
        pltpu.sync_copy(src_hbm.at[pl.ds(eb, CHUNK)], src_v)
        pltpu.sync_copy(dst_hbm.at[pl.ds(eb, CHUNK)], dst_v)
        off = c * N_NODES
        for j in range(CHUNK // 16):
            sl = pl.ds(j * 16, 16)
            src_v[sl] = src_v[sl] + off
            dst_v[sl] = dst_v[sl] + off
        g1 = pltpu.async_copy(H_hbm.at[src_v], rs_v, sem1)
        g2 = pltpu.async_copy(H_hbm.at[dst_v], rd_v, sem2)
        g1.wait()
        g2.wait()

        def _add(i, _):
            for j in range(D // 16):
                sl = pl.ds(j * 16, 16)
                rs_v[i, sl] = rs_v[i, sl] + rd_v[i, sl]
            return 0
        lax.fori_loop(0, CHUNK, _add, 0)

        pltpu.sync_copy(rs_v, msg_hbm.at[pl.ds(c * N_EDGES + eb, CHUNK)])
        return 0

    lax.fori_loop(0, NCH, _chunk, 0)


def _sc_scatter_body(h_hbm, dst_hbm, agg_hbm,
                     dst_v, hv_v, zb_v, sem1, agg_sh):
    c = lax.axis_index("c")
    s = lax.axis_index("s")

    zeros16 = jnp.zeros((16,), F32)

    def _zero_row(i, _):
        for j in range(D // 16):
            zb_v[i, pl.ds(j * 16, 16)] = zeros16
        return 0
    lax.fori_loop(0, CHUNK, _zero_row, 0)
    for r in range(ROWS_PT // CHUNK):
        pltpu.sync_copy(zb_v, agg_sh.at[pl.ds(s * ROWS_PT + r * CHUNK, CHUNK)])

    plsc.subcore_barrier()

    def _chunk(k, _):
        eb = s * EPT + k * CHUNK
        pltpu.sync_copy(dst_hbm.at[pl.ds(eb, CHUNK)], dst_v)
        pltpu.sync_copy(h_hbm.at[pl.ds(c * N_EDGES + eb, CHUNK)], hv_v)
        pltpu.sync_copy(hv_v, agg_sh.at[dst_v], add=True)
        return 0

    lax.fori_loop(0, NCH, _chunk, 0)
    plsc.subcore_barrier()

    pltpu.sync_copy(agg_sh.at[pl.ds(s * ROWS_PT, ROWS_PT)],
                    agg_hbm.at[pl.ds(c * NPAD + s * ROWS_PT, ROWS_PT)])


def _sc_kernels():
    if "g1" not in _sc_cache:
        mesh = plsc.VectorSubcoreMesh(core_axis_name="c", subcore_axis_name="s")
        _sc_cache["g1"] = functools.partial(
            pl.kernel,
            mesh=mesh,
            out_type=jax.ShapeDtypeStruct((N_EDGES, D), F32),
            scratch_types=[
                pltpu.VMEM((CHUNK,), jnp.int32),
                pltpu.VMEM((CHUNK,), jnp.int32),
                pltpu.VMEM((CHUNK, D), F32),
                pltpu.VMEM((CHUNK, D), F32),
                pltpu.SemaphoreType.DMA,
                pltpu.SemaphoreType.DMA,
            ],
        )(_sc_gather1_body)
        _sc_cache["g2"] = functools.partial(
            pl.kernel,
            mesh=mesh,
            out_type=jax.ShapeDtypeStruct((2 * N_EDGES, D), F32),
            scratch_types=[
                pltpu.VMEM((CHUNK,), jnp.int32),
                pltpu.VMEM((CHUNK,), jnp.int32),
                pltpu.VMEM((CHUNK, D), F32),
                pltpu.VMEM((CHUNK, D), F32),
                pltpu.SemaphoreType.DMA,
                pltpu.SemaphoreType.DMA,
            ],
        )(_sc_gather2_body)
        _sc_cache["sc"] = functools.partial(
            pl.kernel,
            mesh=mesh,
            out_type=jax.ShapeDtypeStruct((2 * NPAD, D), F32),
            scratch_types=[
                pltpu.VMEM((CHUNK,), jnp.int32),
                pltpu.VMEM((CHUNK, D), F32),
                pltpu.VMEM((CHUNK, D), F32),
                pltpu.SemaphoreType.DMA,
                pltpu.VMEM_SHARED((NPAD, D), F32),
            ],
        )(_sc_scatter_body)
    return _sc_cache["g1"], _sc_cache["g2"], _sc_cache["sc"]


def _edge1_kernel(gp_ref, msg_ref, tsum_ref, wgp_ref, wn2e_ref, bgp_ref,
                  o_ref):
    msg_ext = jnp.concatenate(
        [msg_ref[...], tsum_ref[0], jnp.zeros((EBLK, 7), F32)], axis=1)
    acc = (jnp.dot(gp_ref[...], wgp_ref[0], preferred_element_type=F32)
           + jnp.dot(msg_ext, wn2e_ref[0], preferred_element_type=F32)
           + bgp_ref[0])
    o_ref[...] = jnp.maximum(acc, 0.0)


def _edge1(gp, msg, tsum, wgp, wn2e, bgp):
    nb = N_EDGES // EBLK
    return pl.pallas_call(
        _edge1_kernel,
        grid=(2, nb),
        in_specs=[
            pl.BlockSpec((EBLK, 16), lambda g, e: (e, 0)),
            pl.BlockSpec((EBLK, D), lambda g, e: (e, 0)),
            pl.BlockSpec((1, EBLK, 1), lambda g, e: (e, 0, 0)),
            pl.BlockSpec((1, 16, D), lambda g, e: (g, 0, 0)),
            pl.BlockSpec((1, 136, D), lambda g, e: (g, 0, 0)),
            pl.BlockSpec((1, 1, D), lambda g, e: (g, 0, 0)),
        ],
        out_specs=pl.BlockSpec((EBLK, D), lambda g, e: (g * nb + e, 0)),
        out_shape=jax.ShapeDtypeStruct((2 * N_EDGES, D), F32),
    )(gp, msg, tsum, wgp, wn2e, bgp)


def _edge2_kernel(h1_ref, msg_ref, wgp_ref, wn2e_ref, bgp_ref, o_ref):
    acc = (jnp.dot(h1_ref[...], wgp_ref[0], preferred_element_type=F32)
           + jnp.dot(msg_ref[...], wn2e_ref[0], preferred_element_type=F32)
           + bgp_ref[0])
    o_ref[...] = jnp.maximum(acc, 0.0)


def _edge2(h1, msg, wgp, wn2e, bgp):
    nb = N_EDGES // EBLK
    io = pl.BlockSpec((EBLK, D), lambda g, e: (g * nb + e, 0))
    return pl.pallas_call(
        _edge2_kernel,
        grid=(2, nb),
        in_specs=[
            io, io,
            pl.BlockSpec((1, D, D), lambda g, e: (g, 0, 0)),
            pl.BlockSpec((1, D, D), lambda g, e: (g, 0, 0)),
            pl.BlockSpec((1, 1, D), lambda g, e: (g, 0, 0)),
        ],
        out_specs=io,
        out_shape=jax.ShapeDtypeStruct((2 * N_EDGES, D), F32),
    )(h1, msg, wgp, wn2e, bgp)


def _tsum_kernel(src_ref, dst_ref, tgt_ref, out_ref):
    def gather(idx):
        oh_lo = jnp.where(
            (idx & 127) == jax.lax.broadcasted_iota(jnp.int32, (EBLK, 128), 1),
            1.0, 0.0)
        m = jnp.dot(oh_lo, tgt_ref[...], preferred_element_type=F32,
                    precision=jax.lax.Precision.HIGHEST)
        oh_hi = jnp.where(
            jax.lax.shift_right_logical(idx, 7)
            == jax.lax.broadcasted_iota(jnp.int32, (EBLK, 128), 1),
            m, 0.0)
        return jnp.sum(oh_hi, axis=1, keepdims=True)

    out_ref[...] = gather(src_ref[0]) + gather(dst_ref[0])


def _tsum(src3, dst3, tgt):
    return pl.pallas_call(
        _tsum_kernel,
        grid=(N_EDGES // EBLK,),
        in_specs=[
            pl.BlockSpec((1, EBLK, 1), lambda e: (e, 0, 0)),
            pl.BlockSpec((1, EBLK, 1), lambda e: (e, 0, 0)),
            pl.BlockSpec((128, 128), lambda e: (0, 0)),
        ],
        out_specs=pl.BlockSpec((EBLK, 1), lambda e: (e, 0)),
        out_shape=jax.ShapeDtypeStruct((N_EDGES, 1), F32),
    )(src3, dst3, tgt)


def _deg_kernel(dst_ref, grid_ref):
    d = dst_ref[0]
    hi = jax.lax.broadcasted_iota(jnp.int32, (EBLK, 128), 1)
    oh_hi = jnp.where(jax.lax.shift_right_logical(d, 7) == hi, 1.0, 0.0)
    oh_lo = jnp.where((d & 127) == hi, 1.0, 0.0)
    part = jax.lax.dot_general(oh_hi, oh_lo, (((0,), (0,)), ((), ())),
                               preferred_element_type=F32)

    @pl.when(pl.program_id(0) == 0)
    def _():
        grid_ref[...] = jnp.zeros_like(grid_ref)

    grid_ref[...] += part


def _deg(dst3):
    return pl.pallas_call(
        _deg_kernel,
        grid=(N_EDGES // EBLK,),
        in_specs=[pl.BlockSpec((1, EBLK, 1), lambda e: (e, 0, 0))],
        out_specs=pl.BlockSpec((128, 128), lambda e: (0, 0)),
        out_shape=jax.ShapeDtypeStruct((128, 128), F32),
    )(dst3)


def _node_upd_kernel(agg_ref, deg_ref, we2n_ref, x_ref, t_ref, wnode_ref,
                     bnode_ref, h2_ref):
    deg = jnp.maximum(deg_ref[...], 1.0)
    aggn = agg_ref[...] / deg
    x_ext = jnp.concatenate(
        [x_ref[...], t_ref[...], jnp.zeros((NBLK, 7), F32)], axis=1)
    h2_ref[...] = jnp.maximum(
        jnp.dot(aggn, we2n_ref[0], preferred_element_type=F32)
        + jnp.dot(x_ext, wnode_ref[0], preferred_element_type=F32)
        + bnode_ref[0], 0.0)


def _node_upd(agg, deg, we2n, x, t, wnode, bnode):
    nb = N_NODES // NBLK
    specs_b = pl.BlockSpec((1, 1, D), lambda g, n: (g, 0, 0))
    io_spec = pl.BlockSpec((NBLK, D), lambda g, n: (g * nb + n, 0))
    return pl.pallas_call(
        _node_upd_kernel,
        grid=(2, nb),
        in_specs=[
            io_spec,
            pl.BlockSpec((NBLK, 1), lambda g, n: (n, 0)),
            pl.BlockSpec((1, D, D), lambda g, n: (g, 0, 0)),
            pl.BlockSpec((NBLK, D), lambda g, n: (n, 0)),
            pl.BlockSpec((NBLK, 1), lambda g, n: (n, 0)),
            pl.BlockSpec((1, 136, D), lambda g, n: (g, 0, 0)),
            specs_b,
        ],
        out_specs=io_spec,
        out_shape=jax.ShapeDtypeStruct((2 * N_NODES, D), F32),
    )(agg, deg, we2n, x, t, wnode, bnode)


def _final_kernel(agg_a_ref, agg_b_ref, deg_ref, h2a_ref, h2b_ref,
                  we2n_a_ref, we2n_b_ref, wn_a_ref, wn_b_ref, bn_a_ref,
                  bn_b_ref, wha_ref, bha_ref, whb_ref, bhb_ref,
                  tan_ref, t_ref, out_ref, sig_ref):
    deg = jnp.maximum(deg_ref[...], 1.0)
    h2a = jnp.maximum(
        jnp.dot(agg_a_ref[...] / deg, we2n_a_ref[...],
                preferred_element_type=F32)
        + jnp.dot(h2a_ref[...], wn_a_ref[...], preferred_element_type=F32)
        + bn_a_ref[...], 0.0)
    h2b = jnp.maximum(
        jnp.dot(agg_b_ref[...] / deg, we2n_b_ref[...],
                preferred_element_type=F32)
        + jnp.dot(h2b_ref[...], wn_b_ref[...], preferred_element_type=F32)
        + bn_b_ref[...], 0.0)
    ha = jnp.dot(h2a, wha_ref[...], preferred_element_type=F32) + bha_ref[...]
    hb = jnp.dot(h2b, whb_ref[...], preferred_element_type=F32) + bhb_ref[...]
    e = jnp.exp(-jnp.abs(ha[:, 2:3]) / t_ref[...])
    bv = jnp.abs(hb[:, 0:1])
    tan = tan_ref[...]
    out_ref[...] = ((ha[:, 0:1] * tan[:, 0:3] + ha[:, 1:2] * tan[:, 3:6])
                    / bv * e)
    sig_ref[...] = jnp.abs(hb[:, 1:2])


def _final(agg_a, agg_b, deg, h2a, h2b, we2n_a, we2n_b, wn_a, wn_b, bn_a,
           bn_b, wha, bha, whb, bhb, tan6, t):
    nb = N_NODES // NBLK
    row = lambda width: pl.BlockSpec((NBLK, width), lambda n: (n, 0))
    full = lambda a, b: pl.BlockSpec((a, b), lambda n: (0, 0))
    return pl.pallas_call(
        _final_kernel,
        grid=(nb,),
        in_specs=[
            row(D), row(D), row(1), row(D), row(D),
            full(D, D), full(D, D), full(D, D), full(D, D),
            full(1, D), full(1, D),
            full(D, 3), full(1, 3), full(D, 2), full(1, 2),
            row(6), row(1),
        ],
        out_specs=[row(3), row(1)],
        out_shape=[jax.ShapeDtypeStruct((N_NODES, 3), F32),
                   jax.ShapeDtypeStruct((N_NODES, 1), F32)],
    )(agg_a, agg_b, deg, h2a, h2b, we2n_a, we2n_b, wn_a, wn_b, bn_a, bn_b,
      wha, bha, whb, bhb, tan6, t)


def _blkdiag(a, b):
    z = jnp.zeros((a.shape[0], b.shape[1]), F32)
    z2 = jnp.zeros((b.shape[0], a.shape[1]), F32)
    return jnp.concatenate([
        jnp.concatenate([a, z], axis=1),
        jnp.concatenate([z2, b], axis=1),
    ], axis=0)


def kernel(feature_GP, feature_Node, feature_tan, feature_T, params,
           edge_index):
    p = params
    src = edge_index[0]
    dst = edge_index[1]

    groups = (("conv1", "conv2", "Econv1", "Econv2"),
              ("Bconv1", "Bconv2", "sigconv1", "sigconv2"))

    w1gp, b1gp, wn2e1, wt1, wnode1, wtnode1, bnode1 = [], [], [], [], [], [], []
    we2n1, wgp2, bgp2, wn2e2, wnode2, bnode2, we2n2 = [], [], [], [], [], [], []
    for gi, (l1a, l2a, l1b, l2b) in enumerate(groups):
        pa1, pa2, pb1, pb2 = p[l1a], p[l2a], p[l1b], p[l2b]
        w1gp.append(jnp.concatenate([pa1["Wgp"], pb1["Wgp"]], axis=1))
        b1gp.append(jnp.concatenate([pa1["bgp"], pb1["bgp"]]))
        if gi == 0:
            wn2e1.append(jnp.concatenate([pa1["Wn2e"], pb1["Wn2e"]], axis=1))
            wt1.append(jnp.zeros((1, D), F32))
            wnode1.append(jnp.concatenate([pa1["Wnode"], pb1["Wnode"]], axis=1))
            wtnode1.append(jnp.zeros((1, D), F32))
        else:
            wn2e1.append(jnp.concatenate(
                [pa1["Wn2e"][:128], pb1["Wn2e"][:128]], axis=1))
            wt1.append(jnp.concatenate(
                [pa1["Wn2e"][128:129], pb1["Wn2e"][128:129]], axis=1))
            wnode1.append(jnp.concatenate(
                [pa1["Wnode"][:128], pb1["Wnode"][:128]], axis=1))
            wtnode1.append(jnp.concatenate(
                [pa1["Wnode"][128:129], pb1["Wnode"][128:129]], axis=1))
        bnode1.append(jnp.concatenate([pa1["bnode"], pb1["bnode"]]))
        we2n1.append(_blkdiag(pa1["We2n"], pb1["We2n"]))
        wgp2.append(_blkdiag(pa2["Wgp"], pb2["Wgp"]))
        bgp2.append(jnp.concatenate([pa2["bgp"], pb2["bgp"]]))
        wn2e2.append(_blkdiag(pa2["Wn2e"], pb2["Wn2e"]))
        wnode2.append(_blkdiag(pa2["Wnode"], pb2["Wnode"]))
        bnode2.append(jnp.concatenate([pa2["bnode"], pb2["bnode"]]))
        we2n2.append(_blkdiag(pa2["We2n"], pb2["We2n"]))

    stk = lambda xs: jnp.stack(xs)
    stkb = lambda xs: jnp.stack(xs).reshape(2, 1, D)

    g1, g2, sc = _sc_kernels()

    nb = N_EDGES // EBLK
    src3 = src.reshape(nb, EBLK, 1)
    dst3 = dst.reshape(nb, EBLK, 1)
    tgrid_t = jnp.pad(feature_T[:, 0], (0, 128 * 128 - N_NODES)) \
        .reshape(128, 128).T
    z7 = jnp.zeros((2, 7, D), F32)
    wn2e1_ext = jnp.concatenate(
        [stk(wn2e1), jnp.stack(wt1).reshape(2, 1, D), z7], axis=1)
    wnode1_ext = jnp.concatenate(
        [stk(wnode1), jnp.stack(wtnode1).reshape(2, 1, D), z7], axis=1)

    msg1 = g1(feature_Node, src, dst)
    tsum = _tsum(src3, dst3, tgrid_t)
    h1c = _edge1(feature_GP, msg1, tsum.reshape(nb, EBLK, 1),
                 stk(w1gp), wn2e1_ext, stkb(b1gp))
    agg1p = sc(h1c, dst)
    agg1 = agg1p.reshape(2, NPAD, D)[:, :N_NODES].reshape(2 * N_NODES, D)
    deg = _deg(dst3).reshape(128 * 128, 1)[:N_NODES]
    h2c = _node_upd(agg1, deg, stk(we2n1), feature_Node, feature_T,
                    wnode1_ext, stkb(bnode1))

    msg2 = g2(h2c, src, dst)
    h1pc = _edge2(h1c, msg2, stk(wgp2), stk(wn2e2), stkb(bgp2))
    agg2p = sc(h1pc, dst)
    agg2 = agg2p.reshape(2, NPAD, D)[:, :N_NODES]

    wha = jnp.zeros((D, 3), F32)
    wha = wha.at[0:64, 0:2].set(p["linear"]["W"])
    wha = wha.at[64:128, 2:3].set(p["Elinear"]["W"])
    bha = jnp.concatenate([p["linear"]["b"], p["Elinear"]["b"]]).reshape(1, 3)
    whb = jnp.zeros((D, 2), F32)
    whb = whb.at[0:64, 0:1].set(p["Blinear"]["W"])
    whb = whb.at[64:128, 1:2].set(p["siglinear"]["W"])
    bhb = jnp.concatenate([p["Blinear"]["b"], p["siglinear"]["b"]]).reshape(1, 2)

    tan6 = feature_tan.reshape(N_NODES, 6)
    out, sig = _final(agg2[0], agg2[1], deg, h2c[:N_NODES], h2c[N_NODES:],
                      we2n2[0], we2n2[1], wnode2[0], wnode2[1],
                      bnode2[0].reshape(1, D), bnode2[1].reshape(1, D),
                      wha, bha, whb, bhb, tan6, feature_T)
    return out, sig


# speedup vs baseline: 4.3925x; 1.1016x over previous
import functools

import jax
import jax.numpy as jnp
from jax import lax
from jax.experimental import pallas as pl
from jax.experimental.pallas import tpu as pltpu
from jax.experimental.pallas import tpu_sc as plsc

F32 = jnp.float32
BF16 = jnp.bfloat16

N_NODES = 10000
N_EDGES = 320000
D = 128
TILES = 16
WORKERS = 32
NPAD = 10240
ROWS_PT = NPAD // TILES
CHUNK = 80

EPT = N_EDGES // TILES
NCH = EPT // CHUNK
EPW = N_EDGES // WORKERS
NCHW = EPW // CHUNK

EBLK = 4000
NBLK = 1000

_sc_cache = {}


def _sc_gather1_body(X_hbm, src_hbm, dst_hbm, msg_hbm,
                     src_v, dst_v, rs_v, rd_v, sem1, sem2):
    c = lax.axis_index("c")
    s = lax.axis_index("s")
    w = c * TILES + s

    def _chunk(k, _):
        eb = w * EPW + k * CHUNK
        pltpu.sync_copy(src_hbm.at[pl.ds(eb, CHUNK)], src_v)
        pltpu.sync_copy(dst_hbm.at[pl.ds(eb, CHUNK)], dst_v)
        g1 = pltpu.async_copy(X_hbm.at[src_v], rs_v, sem1)
        g2 = pltpu.async_copy(X_hbm.at[dst_v], rd_v, sem2)
        g1.wait()
        g2.wait()

        def _add(i, _):
            for j in range(D // 16):
                sl = pl.ds(j * 16, 16)
                rs_v[i, sl] = rs_v[i, sl] + rd_v[i, sl]
            return 0
        lax.fori_loop(0, CHUNK, _add, 0)

        pltpu.sync_copy(rs_v, msg_hbm.at[pl.ds(eb, CHUNK)])
        return 0

    lax.fori_loop(0, NCHW, _chunk, 0)


def _sc_gather2_body(H_hbm, src_hbm, dst_hbm, msg_hbm,
                     src_a, dst_a, rs_a, rd_a, src_b, dst_b, rs_b, rd_b,
                     sem1a, sem2a, sem1b, sem2b):
    c = lax.axis_index("c")
    s = lax.axis_index("s")
    off = c * N_NODES

    def _issue(k, src_v, dst_v, rs_v, rd_v, s1, s2):
        eb = s * EPT + k * CHUNK
        pltpu.sync_copy(src_hbm.at[pl.ds(eb, CHUNK)], src_v)
        pltpu.sync_copy(dst_hbm.at[pl.ds(eb, CHUNK)], dst_v)
        for j in range(CHUNK // 16):
            sl = pl.ds(j * 16, 16)
            src_v[sl] = src_v[sl] + off
            dst_v[sl] = dst_v[sl] + off
        pltpu.async_copy(H_hbm.at[src_v], rs_v, s1)
        pltpu.async_copy(H_hbm.at[dst_v], rd_v, s2)

    def _drain(k, src_v, dst_v, rs_v, rd_v, s1, s2):
        pltpu.make_async_copy(H_hbm.at[src_v], rs_v, s1).wait()
        pltpu.make_async_copy(H_hbm.at[dst_v], rd_v, s2).wait()

        def _add(i, _):
            for j in range(D // 16):
                sl = pl.ds(j * 16, 16)
                rs_v[i, sl] = rs_v[i, sl] + rd_v[i, sl]
            return 0
        lax.fori_loop(0, CHUNK, _add, 0)
        eb = s * EPT + k * CHUNK
        pltpu.sync_copy(rs_v, msg_hbm.at[pl.ds(c * N_EDGES + eb, CHUNK)])

    bufs_a = (src_a, dst_a, rs_a, rd_a, sem1a, sem2a)
    bufs_b = (src_b, dst_b, rs_b, rd_b, sem1b, sem2b)

    _issue(0, *bufs_a)

    def _pair(m, _):
        k0 = 2 * m
        _issue(k0 + 1, *bufs_b)
        _drain(k0, *bufs_a)

        @pl.when(k0 + 2 < NCH)
        def _():
            _issue(k0 + 2, *bufs_a)
        _drain(k0 + 1, *bufs_b)
        return 0

    lax.fori_loop(0, NCH // 2, _pair, 0)


def _sc_scatter_body(h_hbm, dst_hbm, agg_hbm,
                     dst_v, hv_v, zb_v, sem1, agg_sh):
    c = lax.axis_index("c")
    s = lax.axis_index("s")

    zeros16 = jnp.zeros((16,), F32)

    def _zero_row(i, _):
        for j in range(D // 16):
            zb_v[i, pl.ds(j * 16, 16)] = zeros16
        return 0
    lax.fori_loop(0, CHUNK, _zero_row, 0)
    for r in range(ROWS_PT // CHUNK):
        pltpu.sync_copy(zb_v, agg_sh.at[pl.ds(s * ROWS_PT + r * CHUNK, CHUNK)])

    plsc.subcore_barrier()

    def _chunk(k, _):
        eb = s * EPT + k * CHUNK
        pltpu.sync_copy(dst_hbm.at[pl.ds(eb, CHUNK)], dst_v)
        pltpu.sync_copy(h_hbm.at[pl.ds(c * N_EDGES + eb, CHUNK)], hv_v)
        pltpu.sync_copy(hv_v, agg_sh.at[dst_v], add=True)
        return 0

    lax.fori_loop(0, NCH, _chunk, 0)
    plsc.subcore_barrier()

    pltpu.sync_copy(agg_sh.at[pl.ds(s * ROWS_PT, ROWS_PT)],
                    agg_hbm.at[pl.ds(c * NPAD + s * ROWS_PT, ROWS_PT)])


def _sc_kernels():
    if "g1" not in _sc_cache:
        mesh = plsc.VectorSubcoreMesh(core_axis_name="c", subcore_axis_name="s")
        _sc_cache["g1"] = functools.partial(
            pl.kernel,
            mesh=mesh,
            out_type=jax.ShapeDtypeStruct((N_EDGES, D), F32),
            scratch_types=[
                pltpu.VMEM((CHUNK,), jnp.int32),
                pltpu.VMEM((CHUNK,), jnp.int32),
                pltpu.VMEM((CHUNK, D), F32),
                pltpu.VMEM((CHUNK, D), F32),
                pltpu.SemaphoreType.DMA,
                pltpu.SemaphoreType.DMA,
            ],
        )(_sc_gather1_body)
        _sc_cache["g2"] = functools.partial(
            pl.kernel,
            mesh=mesh,
            out_type=jax.ShapeDtypeStruct((2 * N_EDGES, D), F32),
            scratch_types=[
                pltpu.VMEM((CHUNK,), jnp.int32),
                pltpu.VMEM((CHUNK,), jnp.int32),
                pltpu.VMEM((CHUNK, D), F32),
                pltpu.VMEM((CHUNK, D), F32),
                pltpu.VMEM((CHUNK,), jnp.int32),
                pltpu.VMEM((CHUNK,), jnp.int32),
                pltpu.VMEM((CHUNK, D), F32),
                pltpu.VMEM((CHUNK, D), F32),
                pltpu.SemaphoreType.DMA,
                pltpu.SemaphoreType.DMA,
                pltpu.SemaphoreType.DMA,
                pltpu.SemaphoreType.DMA,
            ],
        )(_sc_gather2_body)
        _sc_cache["sc"] = functools.partial(
            pl.kernel,
            mesh=mesh,
            out_type=jax.ShapeDtypeStruct((2 * NPAD, D), F32),
            scratch_types=[
                pltpu.VMEM((CHUNK,), jnp.int32),
                pltpu.VMEM((CHUNK, D), F32),
                pltpu.VMEM((CHUNK, D), F32),
                pltpu.SemaphoreType.DMA,
                pltpu.VMEM_SHARED((NPAD, D), F32),
            ],
        )(_sc_scatter_body)
    return _sc_cache["g1"], _sc_cache["g2"], _sc_cache["sc"]


def _edge1_kernel(gp_ref, msg_ref, tsum_ref, wgp_ref, wn2e_ref, bgp_ref,
                  o_ref):
    msg_ext = jnp.concatenate(
        [msg_ref[...], tsum_ref[0], jnp.zeros((EBLK, 7), F32)], axis=1)
    acc = (jnp.dot(gp_ref[...], wgp_ref[0], preferred_element_type=F32)
           + jnp.dot(msg_ext, wn2e_ref[0], preferred_element_type=F32)
           + bgp_ref[0])
    o_ref[...] = jnp.maximum(acc, 0.0)


def _edge1(gp, msg, tsum, wgp, wn2e, bgp):
    nb = N_EDGES // EBLK
    return pl.pallas_call(
        _edge1_kernel,
        grid=(2, nb),
        in_specs=[
            pl.BlockSpec((EBLK, 16), lambda g, e: (e, 0)),
            pl.BlockSpec((EBLK, D), lambda g, e: (e, 0)),
            pl.BlockSpec((1, EBLK, 1), lambda g, e: (e, 0, 0)),
            pl.BlockSpec((1, 16, D), lambda g, e: (g, 0, 0)),
            pl.BlockSpec((1, 136, D), lambda g, e: (g, 0, 0)),
            pl.BlockSpec((1, 1, D), lambda g, e: (g, 0, 0)),
        ],
        out_specs=pl.BlockSpec((EBLK, D), lambda g, e: (g * nb + e, 0)),
        out_shape=jax.ShapeDtypeStruct((2 * N_EDGES, D), F32),
    )(gp, msg, tsum, wgp, wn2e, bgp)


def _edge2_kernel(h1_ref, msg_ref, wgp_ref, wn2e_ref, bgp_ref, o_ref):
    acc = (jnp.dot(h1_ref[...], wgp_ref[0], preferred_element_type=F32)
           + jnp.dot(msg_ref[...], wn2e_ref[0], preferred_element_type=F32)
           + bgp_ref[0])
    o_ref[...] = jnp.maximum(acc, 0.0)


def _edge2(h1, msg, wgp, wn2e, bgp):
    nb = N_EDGES // EBLK
    io = pl.BlockSpec((EBLK, D), lambda g, e: (g * nb + e, 0))
    return pl.pallas_call(
        _edge2_kernel,
        grid=(2, nb),
        in_specs=[
            io, io,
            pl.BlockSpec((1, D, D), lambda g, e: (g, 0, 0)),
            pl.BlockSpec((1, D, D), lambda g, e: (g, 0, 0)),
            pl.BlockSpec((1, 1, D), lambda g, e: (g, 0, 0)),
        ],
        out_specs=io,
        out_shape=jax.ShapeDtypeStruct((2 * N_EDGES, D), F32),
    )(h1, msg, wgp, wn2e, bgp)


def _tsum_kernel(src_ref, dst_ref, tgt_ref, out_ref):
    def gather(idx):
        oh_lo = jnp.where(
            (idx & 127) == jax.lax.broadcasted_iota(jnp.int32, (EBLK, 128), 1),
            1.0, 0.0)
        m = jnp.dot(oh_lo, tgt_ref[...], preferred_element_type=F32,
                    precision=jax.lax.Precision.HIGHEST)
        oh_hi = jnp.where(
            jax.lax.shift_right_logical(idx, 7)
            == jax.lax.broadcasted_iota(jnp.int32, (EBLK, 128), 1),
            m, 0.0)
        return jnp.sum(oh_hi, axis=1, keepdims=True)

    out_ref[...] = gather(src_ref[0]) + gather(dst_ref[0])


def _tsum(src3, dst3, tgt):
    return pl.pallas_call(
        _tsum_kernel,
        grid=(N_EDGES // EBLK,),
        in_specs=[
            pl.BlockSpec((1, EBLK, 1), lambda e: (e, 0, 0)),
            pl.BlockSpec((1, EBLK, 1), lambda e: (e, 0, 0)),
            pl.BlockSpec((128, 128), lambda e: (0, 0)),
        ],
        out_specs=pl.BlockSpec((EBLK, 1), lambda e: (e, 0)),
        out_shape=jax.ShapeDtypeStruct((N_EDGES, 1), F32),
    )(src3, dst3, tgt)


def _deg_kernel(dst_ref, grid_ref):
    d = dst_ref[0]
    hi = jax.lax.broadcasted_iota(jnp.int32, (EBLK, 128), 1)
    oh_hi = jnp.where(jax.lax.shift_right_logical(d, 7) == hi, 1.0, 0.0)
    oh_lo = jnp.where((d & 127) == hi, 1.0, 0.0)
    part = jax.lax.dot_general(oh_hi, oh_lo, (((0,), (0,)), ((), ())),
                               preferred_element_type=F32)

    @pl.when(pl.program_id(0) == 0)
    def _():
        grid_ref[...] = jnp.zeros_like(grid_ref)

    grid_ref[...] += part


def _deg(dst3):
    return pl.pallas_call(
        _deg_kernel,
        grid=(N_EDGES // EBLK,),
        in_specs=[pl.BlockSpec((1, EBLK, 1), lambda e: (e, 0, 0))],
        out_specs=pl.BlockSpec((128, 128), lambda e: (0, 0)),
        out_shape=jax.ShapeDtypeStruct((128, 128), F32),
    )(dst3)


def _node_upd_kernel(agg_ref, deg_ref, we2n_ref, x_ref, t_ref, wnode_ref,
                     bnode_ref, h2_ref):
    deg = jnp.maximum(deg_ref[...], 1.0)
    aggn = agg_ref[...] / deg
    x_ext = jnp.concatenate(
        [x_ref[...], t_ref[...], jnp.zeros((NBLK, 7), F32)], axis=1)
    h2_ref[...] = jnp.maximum(
        jnp.dot(aggn, we2n_ref[0], preferred_element_type=F32)
        + jnp.dot(x_ext, wnode_ref[0], preferred_element_type=F32)
        + bnode_ref[0], 0.0)


def _node_upd(agg, deg, we2n, x, t, wnode, bnode):
    nb = N_NODES // NBLK
    specs_b = pl.BlockSpec((1, 1, D), lambda g, n: (g, 0, 0))
    io_spec = pl.BlockSpec((NBLK, D), lambda g, n: (g * nb + n, 0))
    return pl.pallas_call(
        _node_upd_kernel,
        grid=(2, nb),
        in_specs=[
            io_spec,
            pl.BlockSpec((NBLK, 1), lambda g, n: (n, 0)),
            pl.BlockSpec((1, D, D), lambda g, n: (g, 0, 0)),
            pl.BlockSpec((NBLK, D), lambda g, n: (n, 0)),
            pl.BlockSpec((NBLK, 1), lambda g, n: (n, 0)),
            pl.BlockSpec((1, 136, D), lambda g, n: (g, 0, 0)),
            specs_b,
        ],
        out_specs=io_spec,
        out_shape=jax.ShapeDtypeStruct((2 * N_NODES, D), F32),
    )(agg, deg, we2n, x, t, wnode, bnode)


def _final_kernel(agg_a_ref, agg_b_ref, deg_ref, h2a_ref, h2b_ref,
                  we2n_a_ref, we2n_b_ref, wn_a_ref, wn_b_ref, bn_a_ref,
                  bn_b_ref, wha_ref, bha_ref, whb_ref, bhb_ref,
                  tan_ref, t_ref, out_ref, sig_ref):
    deg = jnp.maximum(deg_ref[...], 1.0)
    h2a = jnp.maximum(
        jnp.dot(agg_a_ref[...] / deg, we2n_a_ref[...],
                preferred_element_type=F32)
        + jnp.dot(h2a_ref[...], wn_a_ref[...], preferred_element_type=F32)
        + bn_a_ref[...], 0.0)
    h2b = jnp.maximum(
        jnp.dot(agg_b_ref[...] / deg, we2n_b_ref[...],
                preferred_element_type=F32)
        + jnp.dot(h2b_ref[...], wn_b_ref[...], preferred_element_type=F32)
        + bn_b_ref[...], 0.0)
    ha = jnp.dot(h2a, wha_ref[...], preferred_element_type=F32) + bha_ref[...]
    hb = jnp.dot(h2b, whb_ref[...], preferred_element_type=F32) + bhb_ref[...]
    e = jnp.exp(-jnp.abs(ha[:, 2:3]) / t_ref[...])
    bv = jnp.abs(hb[:, 0:1])
    tan = tan_ref[...]
    out_ref[...] = ((ha[:, 0:1] * tan[:, 0:3] + ha[:, 1:2] * tan[:, 3:6])
                    / bv * e)
    sig_ref[...] = jnp.abs(hb[:, 1:2])


def _final(agg_a, agg_b, deg, h2a, h2b, we2n_a, we2n_b, wn_a, wn_b, bn_a,
           bn_b, wha, bha, whb, bhb, tan6, t):
    nb = N_NODES // NBLK
    row = lambda width: pl.BlockSpec((NBLK, width), lambda n: (n, 0))
    full = lambda a, b: pl.BlockSpec((a, b), lambda n: (0, 0))
    return pl.pallas_call(
        _final_kernel,
        grid=(nb,),
        in_specs=[
            row(D), row(D), row(1), row(D), row(D),
            full(D, D), full(D, D), full(D, D), full(D, D),
            full(1, D), full(1, D),
            full(D, 3), full(1, 3), full(D, 2), full(1, 2),
            row(6), row(1),
        ],
        out_specs=[row(3), row(1)],
        out_shape=[jax.ShapeDtypeStruct((N_NODES, 3), F32),
                   jax.ShapeDtypeStruct((N_NODES, 1), F32)],
    )(agg_a, agg_b, deg, h2a, h2b, we2n_a, we2n_b, wn_a, wn_b, bn_a, bn_b,
      wha, bha, whb, bhb, tan6, t)


def _blkdiag(a, b):
    z = jnp.zeros((a.shape[0], b.shape[1]), F32)
    z2 = jnp.zeros((b.shape[0], a.shape[1]), F32)
    return jnp.concatenate([
        jnp.concatenate([a, z], axis=1),
        jnp.concatenate([z2, b], axis=1),
    ], axis=0)


def kernel(feature_GP, feature_Node, feature_tan, feature_T, params,
           edge_index):
    p = params
    src = edge_index[0]
    dst = edge_index[1]

    groups = (("conv1", "conv2", "Econv1", "Econv2"),
              ("Bconv1", "Bconv2", "sigconv1", "sigconv2"))

    w1gp, b1gp, wn2e1, wt1, wnode1, wtnode1, bnode1 = [], [], [], [], [], [], []
    we2n1, wgp2, bgp2, wn2e2, wnode2, bnode2, we2n2 = [], [], [], [], [], [], []
    for gi, (l1a, l2a, l1b, l2b) in enumerate(groups):
        pa1, pa2, pb1, pb2 = p[l1a], p[l2a], p[l1b], p[l2b]
        w1gp.append(jnp.concatenate([pa1["Wgp"], pb1["Wgp"]], axis=1))
        b1gp.append(jnp.concatenate([pa1["bgp"], pb1["bgp"]]))
        if gi == 0:
            wn2e1.append(jnp.concatenate([pa1["Wn2e"], pb1["Wn2e"]], axis=1))
            wt1.append(jnp.zeros((1, D), F32))
            wnode1.append(jnp.concatenate([pa1["Wnode"], pb1["Wnode"]], axis=1))
            wtnode1.append(jnp.zeros((1, D), F32))
        else:
            wn2e1.append(jnp.concatenate(
                [pa1["Wn2e"][:128], pb1["Wn2e"][:128]], axis=1))
            wt1.append(jnp.concatenate(
                [pa1["Wn2e"][128:129], pb1["Wn2e"][128:129]], axis=1))
            wnode1.append(jnp.concatenate(
                [pa1["Wnode"][:128], pb1["Wnode"][:128]], axis=1))
            wtnode1.append(jnp.concatenate(
                [pa1["Wnode"][128:129], pb1["Wnode"][128:129]], axis=1))
        bnode1.append(jnp.concatenate([pa1["bnode"], pb1["bnode"]]))
        we2n1.append(_blkdiag(pa1["We2n"], pb1["We2n"]))
        wgp2.append(_blkdiag(pa2["Wgp"], pb2["Wgp"]))
        bgp2.append(jnp.concatenate([pa2["bgp"], pb2["bgp"]]))
        wn2e2.append(_blkdiag(pa2["Wn2e"], pb2["Wn2e"]))
        wnode2.append(_blkdiag(pa2["Wnode"], pb2["Wnode"]))
        bnode2.append(jnp.concatenate([pa2["bnode"], pb2["bnode"]]))
        we2n2.append(_blkdiag(pa2["We2n"], pb2["We2n"]))

    stk = lambda xs: jnp.stack(xs)
    stkb = lambda xs: jnp.stack(xs).reshape(2, 1, D)

    g1, g2, sc = _sc_kernels()

    nb = N_EDGES // EBLK
    src3 = src.reshape(nb, EBLK, 1)
    dst3 = dst.reshape(nb, EBLK, 1)
    tgrid_t = jnp.pad(feature_T[:, 0], (0, 128 * 128 - N_NODES)) \
        .reshape(128, 128).T
    z7 = jnp.zeros((2, 7, D), F32)
    wn2e1_ext = jnp.concatenate(
        [stk(wn2e1), jnp.stack(wt1).reshape(2, 1, D), z7], axis=1)
    wnode1_ext = jnp.concatenate(
        [stk(wnode1), jnp.stack(wtnode1).reshape(2, 1, D), z7], axis=1)

    msg1 = g1(feature_Node, src, dst)
    tsum = _tsum(src3, dst3, tgrid_t)
    h1c = _edge1(feature_GP, msg1, tsum.reshape(nb, EBLK, 1),
                 stk(w1gp), wn2e1_ext, stkb(b1gp))
    agg1p = sc(h1c, dst)
    agg1 = agg1p.reshape(2, NPAD, D)[:, :N_NODES].reshape(2 * N_NODES, D)
    deg = _deg(dst3).reshape(128 * 128, 1)[:N_NODES]
    h2c = _node_upd(agg1, deg, stk(we2n1), feature_Node, feature_T,
                    wnode1_ext, stkb(bnode1))

    msg2 = g2(h2c, src, dst)
    h1pc = _edge2(h1c, msg2, stk(wgp2), stk(wn2e2), stkb(bgp2))
    agg2p = sc(h1pc, dst)
    agg2 = agg2p.reshape(2, NPAD, D)[:, :N_NODES]

    wha = jnp.zeros((D, 3), F32)
    wha = wha.at[0:64, 0:2].set(p["linear"]["W"])
    wha = wha.at[64:128, 2:3].set(p["Elinear"]["W"])
    bha = jnp.concatenate([p["linear"]["b"], p["Elinear"]["b"]]).reshape(1, 3)
    whb = jnp.zeros((D, 2), F32)
    whb = whb.at[0:64, 0:1].set(p["Blinear"]["W"])
    whb = whb.at[64:128, 1:2].set(p["siglinear"]["W"])
    bhb = jnp.concatenate([p["Blinear"]["b"], p["siglinear"]["b"]]).reshape(1, 2)

    tan6 = feature_tan.reshape(N_NODES, 6)
    out, sig = _final(agg2[0], agg2[1], deg, h2c[:N_NODES], h2c[N_NODES:],
                      we2n2[0], we2n2[1], wnode2[0], wnode2[1],
                      bnode2[0].reshape(1, D), bnode2[1].reshape(1, D),
                      wha, bha, whb, bhb, tan6, feature_T)
    return out, sig


# speedup vs baseline: 5.2526x; 1.1958x over previous
import functools

import jax
import jax.numpy as jnp
from jax import lax
from jax.experimental import pallas as pl
from jax.experimental.pallas import tpu as pltpu
from jax.experimental.pallas import tpu_sc as plsc

F32 = jnp.float32
BF16 = jnp.bfloat16

N_NODES = 10000
N_EDGES = 320000
D = 128
TILES = 16
WORKERS = 32
NPAD = 10240
ROWS_PT = NPAD // TILES
CHUNK = 80

EPT = N_EDGES // TILES
NCH = EPT // CHUNK
EPW = N_EDGES // WORKERS
NCHW = EPW // CHUNK

EBLK = 4000
NBLK = 1000

_sc_cache = {}


def _sc_gather1_body(X_hbm, src_hbm, dst_hbm, msg_hbm,
                     src_a, dst_a, rs_a, rd_a, src_b, dst_b, rs_b, rd_b,
                     sem1a, sem2a, sem1b, sem2b):
    c = lax.axis_index("c")
    s = lax.axis_index("s")
    w = c * TILES + s

    def _issue(k, src_v, dst_v, rs_v, rd_v, s1, s2):
        eb = w * EPW + k * CHUNK
        pltpu.sync_copy(src_hbm.at[pl.ds(eb, CHUNK)], src_v)
        pltpu.sync_copy(dst_hbm.at[pl.ds(eb, CHUNK)], dst_v)
        pltpu.async_copy(X_hbm.at[src_v], rs_v, s1)
        pltpu.async_copy(X_hbm.at[dst_v], rd_v, s2)

    def _drain(k, src_v, dst_v, rs_v, rd_v, s1, s2):
        pltpu.make_async_copy(X_hbm.at[src_v], rs_v, s1).wait()
        pltpu.make_async_copy(X_hbm.at[dst_v], rd_v, s2).wait()

        def _add(i, _):
            for j in range(D // 16):
                sl = pl.ds(j * 16, 16)
                rs_v[i, sl] = rs_v[i, sl] + rd_v[i, sl]
            return 0
        lax.fori_loop(0, CHUNK, _add, 0)
        eb = w * EPW + k * CHUNK
        pltpu.sync_copy(rs_v, msg_hbm.at[pl.ds(eb, CHUNK)])

    bufs_a = (src_a, dst_a, rs_a, rd_a, sem1a, sem2a)
    bufs_b = (src_b, dst_b, rs_b, rd_b, sem1b, sem2b)

    _issue(0, *bufs_a)

    def _pair(m, _):
        k0 = 2 * m
        _issue(k0 + 1, *bufs_b)
        _drain(k0, *bufs_a)
        _issue(k0 + 2, *bufs_a)
        _drain(k0 + 1, *bufs_b)
        return 0

    lax.fori_loop(0, NCHW // 2, _pair, 0)
    _drain(NCHW - 1, *bufs_a)


def _sc_gather2_body(H_hbm, src_hbm, dst_hbm, msg_hbm,
                     src_a, dst_a, rs_a, rd_a, src_b, dst_b, rs_b, rd_b,
                     sem1a, sem2a, sem1b, sem2b):
    c = lax.axis_index("c")
    s = lax.axis_index("s")
    off = c * N_NODES

    def _issue(k, src_v, dst_v, rs_v, rd_v, s1, s2):
        eb = s * EPT + k * CHUNK
        pltpu.sync_copy(src_hbm.at[pl.ds(eb, CHUNK)], src_v)
        pltpu.sync_copy(dst_hbm.at[pl.ds(eb, CHUNK)], dst_v)
        for j in range(CHUNK // 16):
            sl = pl.ds(j * 16, 16)
            src_v[sl] = src_v[sl] + off
            dst_v[sl] = dst_v[sl] + off
        pltpu.async_copy(H_hbm.at[src_v], rs_v, s1)
        pltpu.async_copy(H_hbm.at[dst_v], rd_v, s2)

    def _drain(k, src_v, dst_v, rs_v, rd_v, s1, s2):
        pltpu.make_async_copy(H_hbm.at[src_v], rs_v, s1).wait()
        pltpu.make_async_copy(H_hbm.at[dst_v], rd_v, s2).wait()

        def _add(i, _):
            for j in range(D // 16):
                sl = pl.ds(j * 16, 16)
                rs_v[i, sl] = rs_v[i, sl] + rd_v[i, sl]
            return 0
        lax.fori_loop(0, CHUNK, _add, 0)
        eb = s * EPT + k * CHUNK
        pltpu.sync_copy(rs_v, msg_hbm.at[pl.ds(c * N_EDGES + eb, CHUNK)])

    bufs_a = (src_a, dst_a, rs_a, rd_a, sem1a, sem2a)
    bufs_b = (src_b, dst_b, rs_b, rd_b, sem1b, sem2b)

    _issue(0, *bufs_a)

    def _pair(m, _):
        k0 = 2 * m
        _issue(k0 + 1, *bufs_b)
        _drain(k0, *bufs_a)

        @pl.when(k0 + 2 < NCH)
        def _():
            _issue(k0 + 2, *bufs_a)
        _drain(k0 + 1, *bufs_b)
        return 0

    lax.fori_loop(0, NCH // 2, _pair, 0)


def _sc_scatter_body(h_hbm, dst_hbm, agg_hbm,
                     dst_a, hv_a, dst_b, hv_b, sem1a, sem2a, sem1b, sem2b,
                     agg_sh):
    c = lax.axis_index("c")
    s = lax.axis_index("s")

    zeros16 = jnp.zeros((16,), F32)

    def _zero_row(i, _):
        for j in range(D // 16):
            hv_a[i, pl.ds(j * 16, 16)] = zeros16
        return 0
    lax.fori_loop(0, CHUNK, _zero_row, 0)
    for r in range(ROWS_PT // CHUNK):
        pltpu.sync_copy(hv_a, agg_sh.at[pl.ds(s * ROWS_PT + r * CHUNK, CHUNK)])

    plsc.subcore_barrier()

    def _issue(k, dst_v, hv_v, s1, s2):
        eb = s * EPT + k * CHUNK
        pltpu.async_copy(dst_hbm.at[pl.ds(eb, CHUNK)], dst_v, s1)
        pltpu.async_copy(h_hbm.at[pl.ds(c * N_EDGES + eb, CHUNK)], hv_v, s2)

    def _drain(k, dst_v, hv_v, s1, s2):
        eb = s * EPT + k * CHUNK
        pltpu.make_async_copy(dst_hbm.at[pl.ds(eb, CHUNK)], dst_v, s1).wait()
        pltpu.make_async_copy(h_hbm.at[pl.ds(c * N_EDGES + eb, CHUNK)],
                              hv_v, s2).wait()
        pltpu.sync_copy(hv_v, agg_sh.at[dst_v], add=True)

    bufs_a = (dst_a, hv_a, sem1a, sem2a)
    bufs_b = (dst_b, hv_b, sem1b, sem2b)

    _issue(0, *bufs_a)

    def _pair(m, _):
        k0 = 2 * m
        _issue(k0 + 1, *bufs_b)
        _drain(k0, *bufs_a)

        @pl.when(k0 + 2 < NCH)
        def _():
            _issue(k0 + 2, *bufs_a)
        _drain(k0 + 1, *bufs_b)
        return 0

    lax.fori_loop(0, NCH // 2, _pair, 0)
    plsc.subcore_barrier()

    pltpu.sync_copy(agg_sh.at[pl.ds(s * ROWS_PT, ROWS_PT)],
                    agg_hbm.at[pl.ds(c * NPAD + s * ROWS_PT, ROWS_PT)])


def _sc_kernels():
    if "g1" not in _sc_cache:
        mesh = plsc.VectorSubcoreMesh(core_axis_name="c", subcore_axis_name="s")
        _sc_cache["g1"] = functools.partial(
            pl.kernel,
            mesh=mesh,
            out_type=jax.ShapeDtypeStruct((N_EDGES, D), F32),
            scratch_types=[
                pltpu.VMEM((CHUNK,), jnp.int32),
                pltpu.VMEM((CHUNK,), jnp.int32),
                pltpu.VMEM((CHUNK, D), F32),
                pltpu.VMEM((CHUNK, D), F32),
                pltpu.VMEM((CHUNK,), jnp.int32),
                pltpu.VMEM((CHUNK,), jnp.int32),
                pltpu.VMEM((CHUNK, D), F32),
                pltpu.VMEM((CHUNK, D), F32),
                pltpu.SemaphoreType.DMA,
                pltpu.SemaphoreType.DMA,
                pltpu.SemaphoreType.DMA,
                pltpu.SemaphoreType.DMA,
            ],
        )(_sc_gather1_body)
        _sc_cache["g2"] = functools.partial(
            pl.kernel,
            mesh=mesh,
            out_type=jax.ShapeDtypeStruct((2 * N_EDGES, D), F32),
            scratch_types=[
                pltpu.VMEM((CHUNK,), jnp.int32),
                pltpu.VMEM((CHUNK,), jnp.int32),
                pltpu.VMEM((CHUNK, D), F32),
                pltpu.VMEM((CHUNK, D), F32),
                pltpu.VMEM((CHUNK,), jnp.int32),
                pltpu.VMEM((CHUNK,), jnp.int32),
                pltpu.VMEM((CHUNK, D), F32),
                pltpu.VMEM((CHUNK, D), F32),
                pltpu.SemaphoreType.DMA,
                pltpu.SemaphoreType.DMA,
                pltpu.SemaphoreType.DMA,
                pltpu.SemaphoreType.DMA,
            ],
        )(_sc_gather2_body)
        _sc_cache["sc"] = functools.partial(
            pl.kernel,
            mesh=mesh,
            out_type=jax.ShapeDtypeStruct((2 * NPAD, D), F32),
            scratch_types=[
                pltpu.VMEM((CHUNK,), jnp.int32),
                pltpu.VMEM((CHUNK, D), F32),
                pltpu.VMEM((CHUNK,), jnp.int32),
                pltpu.VMEM((CHUNK, D), F32),
                pltpu.SemaphoreType.DMA,
                pltpu.SemaphoreType.DMA,
                pltpu.SemaphoreType.DMA,
                pltpu.SemaphoreType.DMA,
                pltpu.VMEM_SHARED((NPAD, D), F32),
            ],
        )(_sc_scatter_body)
    return _sc_cache["g1"], _sc_cache["g2"], _sc_cache["sc"]


def _edge1_kernel(gp_ref, msg_ref, tsum_ref, wgp_ref, wn2e_ref, bgp_ref,
                  o_ref):
    msg_ext = jnp.concatenate(
        [msg_ref[...], tsum_ref[0], jnp.zeros((EBLK, 7), F32)], axis=1)
    acc = (jnp.dot(gp_ref[...], wgp_ref[0], preferred_element_type=F32)
           + jnp.dot(msg_ext, wn2e_ref[0], preferred_element_type=F32)
           + bgp_ref[0])
    o_ref[...] = jnp.maximum(acc, 0.0)


def _edge1(gp, msg, tsum, wgp, wn2e, bgp):
    nb = N_EDGES // EBLK
    return pl.pallas_call(
        _edge1_kernel,
        grid=(2, nb),
        in_specs=[
            pl.BlockSpec((EBLK, 16), lambda g, e: (e, 0)),
            pl.BlockSpec((EBLK, D), lambda g, e: (e, 0)),
            pl.BlockSpec((1, EBLK, 1), lambda g, e: (e, 0, 0)),
            pl.BlockSpec((1, 16, D), lambda g, e: (g, 0, 0)),
            pl.BlockSpec((1, 136, D), lambda g, e: (g, 0, 0)),
            pl.BlockSpec((1, 1, D), lambda g, e: (g, 0, 0)),
        ],
        out_specs=pl.BlockSpec((EBLK, D), lambda g, e: (g * nb + e, 0)),
        out_shape=jax.ShapeDtypeStruct((2 * N_EDGES, D), F32),
    )(gp, msg, tsum, wgp, wn2e, bgp)


def _edge2_kernel(h1_ref, msg_ref, wgp_ref, wn2e_ref, bgp_ref, o_ref):
    acc = (jnp.dot(h1_ref[...], wgp_ref[0], preferred_element_type=F32)
           + jnp.dot(msg_ref[...], wn2e_ref[0], preferred_element_type=F32)
           + bgp_ref[0])
    o_ref[...] = jnp.maximum(acc, 0.0)


def _edge2(h1, msg, wgp, wn2e, bgp):
    nb = N_EDGES // EBLK
    io = pl.BlockSpec((EBLK, D), lambda g, e: (g * nb + e, 0))
    return pl.pallas_call(
        _edge2_kernel,
        grid=(2, nb),
        in_specs=[
            io, io,
            pl.BlockSpec((1, D, D), lambda g, e: (g, 0, 0)),
            pl.BlockSpec((1, D, D), lambda g, e: (g, 0, 0)),
            pl.BlockSpec((1, 1, D), lambda g, e: (g, 0, 0)),
        ],
        out_specs=io,
        out_shape=jax.ShapeDtypeStruct((2 * N_EDGES, D), F32),
    )(h1, msg, wgp, wn2e, bgp)


def _tsum_kernel(src_ref, dst_ref, tgt_ref, out_ref):
    def gather(idx):
        oh_lo = jnp.where(
            (idx & 127) == jax.lax.broadcasted_iota(jnp.int32, (EBLK, 128), 1),
            1.0, 0.0)
        m = jnp.dot(oh_lo, tgt_ref[...], preferred_element_type=F32,
                    precision=jax.lax.Precision.HIGHEST)
        oh_hi = jnp.where(
            jax.lax.shift_right_logical(idx, 7)
            == jax.lax.broadcasted_iota(jnp.int32, (EBLK, 128), 1),
            m, 0.0)
        return jnp.sum(oh_hi, axis=1, keepdims=True)

    out_ref[...] = gather(src_ref[0]) + gather(dst_ref[0])


def _tsum(src3, dst3, tgt):
    return pl.pallas_call(
        _tsum_kernel,
        grid=(N_EDGES // EBLK,),
        in_specs=[
            pl.BlockSpec((1, EBLK, 1), lambda e: (e, 0, 0)),
            pl.BlockSpec((1, EBLK, 1), lambda e: (e, 0, 0)),
            pl.BlockSpec((128, 128), lambda e: (0, 0)),
        ],
        out_specs=pl.BlockSpec((EBLK, 1), lambda e: (e, 0)),
        out_shape=jax.ShapeDtypeStruct((N_EDGES, 1), F32),
    )(src3, dst3, tgt)


def _deg_kernel(dst_ref, grid_ref):
    d = dst_ref[0]
    hi = jax.lax.broadcasted_iota(jnp.int32, (EBLK, 128), 1)
    oh_hi = jnp.where(jax.lax.shift_right_logical(d, 7) == hi, 1.0, 0.0)
    oh_lo = jnp.where((d & 127) == hi, 1.0, 0.0)
    part = jax.lax.dot_general(oh_hi, oh_lo, (((0,), (0,)), ((), ())),
                               preferred_element_type=F32)

    @pl.when(pl.program_id(0) == 0)
    def _():
        grid_ref[...] = jnp.zeros_like(grid_ref)

    grid_ref[...] += part


def _deg(dst3):
    return pl.pallas_call(
        _deg_kernel,
        grid=(N_EDGES // EBLK,),
        in_specs=[pl.BlockSpec((1, EBLK, 1), lambda e: (e, 0, 0))],
        out_specs=pl.BlockSpec((128, 128), lambda e: (0, 0)),
        out_shape=jax.ShapeDtypeStruct((128, 128), F32),
    )(dst3)


def _node_upd_kernel(agg_ref, deg_ref, we2n_ref, x_ref, t_ref, wnode_ref,
                     bnode_ref, h2_ref):
    deg = jnp.maximum(deg_ref[...], 1.0)
    aggn = agg_ref[...] / deg
    x_ext = jnp.concatenate(
        [x_ref[...], t_ref[...], jnp.zeros((NBLK, 7), F32)], axis=1)
    h2_ref[...] = jnp.maximum(
        jnp.dot(aggn, we2n_ref[0], preferred_element_type=F32)
        + jnp.dot(x_ext, wnode_ref[0], preferred_element_type=F32)
        + bnode_ref[0], 0.0)


def _node_upd(agg, deg, we2n, x, t, wnode, bnode):
    nb = N_NODES // NBLK
    specs_b = pl.BlockSpec((1, 1, D), lambda g, n: (g, 0, 0))
    io_spec = pl.BlockSpec((NBLK, D), lambda g, n: (g * nb + n, 0))
    return pl.pallas_call(
        _node_upd_kernel,
        grid=(2, nb),
        in_specs=[
            io_spec,
            pl.BlockSpec((NBLK, 1), lambda g, n: (n, 0)),
            pl.BlockSpec((1, D, D), lambda g, n: (g, 0, 0)),
            pl.BlockSpec((NBLK, D), lambda g, n: (n, 0)),
            pl.BlockSpec((NBLK, 1), lambda g, n: (n, 0)),
            pl.BlockSpec((1, 136, D), lambda g, n: (g, 0, 0)),
            specs_b,
        ],
        out_specs=io_spec,
        out_shape=jax.ShapeDtypeStruct((2 * N_NODES, D), F32),
    )(agg, deg, we2n, x, t, wnode, bnode)


def _final_kernel(agg_a_ref, agg_b_ref, deg_ref, h2a_ref, h2b_ref,
                  we2n_a_ref, we2n_b_ref, wn_a_ref, wn_b_ref, bn_a_ref,
                  bn_b_ref, wha_ref, bha_ref, whb_ref, bhb_ref,
                  tan_ref, t_ref, out_ref, sig_ref):
    deg = jnp.maximum(deg_ref[...], 1.0)
    h2a = jnp.maximum(
        jnp.dot(agg_a_ref[...] / deg, we2n_a_ref[...],
                preferred_element_type=F32)
        + jnp.dot(h2a_ref[...], wn_a_ref[...], preferred_element_type=F32)
        + bn_a_ref[...], 0.0)
    h2b = jnp.maximum(
        jnp.dot(agg_b_ref[...] / deg, we2n_b_ref[...],
                preferred_element_type=F32)
        + jnp.dot(h2b_ref[...], wn_b_ref[...], preferred_element_type=F32)
        + bn_b_ref[...], 0.0)
    ha = jnp.dot(h2a, wha_ref[...], preferred_element_type=F32) + bha_ref[...]
    hb = jnp.dot(h2b, whb_ref[...], preferred_element_type=F32) + bhb_ref[...]
    e = jnp.exp(-jnp.abs(ha[:, 2:3]) / t_ref[...])
    bv = jnp.abs(hb[:, 0:1])
    tan = tan_ref[...]
    out_ref[...] = ((ha[:, 0:1] * tan[:, 0:3] + ha[:, 1:2] * tan[:, 3:6])
                    / bv * e)
    sig_ref[...] = jnp.abs(hb[:, 1:2])


def _final(agg_a, agg_b, deg, h2a, h2b, we2n_a, we2n_b, wn_a, wn_b, bn_a,
           bn_b, wha, bha, whb, bhb, tan6, t):
    nb = N_NODES // NBLK
    row = lambda width: pl.BlockSpec((NBLK, width), lambda n: (n, 0))
    full = lambda a, b: pl.BlockSpec((a, b), lambda n: (0, 0))
    return pl.pallas_call(
        _final_kernel,
        grid=(nb,),
        in_specs=[
            row(D), row(D), row(1), row(D), row(D),
            full(D, D), full(D, D), full(D, D), full(D, D),
            full(1, D), full(1, D),
            full(D, 3), full(1, 3), full(D, 2), full(1, 2),
            row(6), row(1),
        ],
        out_specs=[row(3), row(1)],
        out_shape=[jax.ShapeDtypeStruct((N_NODES, 3), F32),
                   jax.ShapeDtypeStruct((N_NODES, 1), F32)],
    )(agg_a, agg_b, deg, h2a, h2b, we2n_a, we2n_b, wn_a, wn_b, bn_a, bn_b,
      wha, bha, whb, bhb, tan6, t)


def _blkdiag(a, b):
    z = jnp.zeros((a.shape[0], b.shape[1]), F32)
    z2 = jnp.zeros((b.shape[0], a.shape[1]), F32)
    return jnp.concatenate([
        jnp.concatenate([a, z], axis=1),
        jnp.concatenate([z2, b], axis=1),
    ], axis=0)


def kernel(feature_GP, feature_Node, feature_tan, feature_T, params,
           edge_index):
    p = params
    src = edge_index[0]
    dst = edge_index[1]

    groups = (("conv1", "conv2", "Econv1", "Econv2"),
              ("Bconv1", "Bconv2", "sigconv1", "sigconv2"))

    w1gp, b1gp, wn2e1, wt1, wnode1, wtnode1, bnode1 = [], [], [], [], [], [], []
    we2n1, wgp2, bgp2, wn2e2, wnode2, bnode2, we2n2 = [], [], [], [], [], [], []
    for gi, (l1a, l2a, l1b, l2b) in enumerate(groups):
        pa1, pa2, pb1, pb2 = p[l1a], p[l2a], p[l1b], p[l2b]
        w1gp.append(jnp.concatenate([pa1["Wgp"], pb1["Wgp"]], axis=1))
        b1gp.append(jnp.concatenate([pa1["bgp"], pb1["bgp"]]))
        if gi == 0:
            wn2e1.append(jnp.concatenate([pa1["Wn2e"], pb1["Wn2e"]], axis=1))
            wt1.append(jnp.zeros((1, D), F32))
            wnode1.append(jnp.concatenate([pa1["Wnode"], pb1["Wnode"]], axis=1))
            wtnode1.append(jnp.zeros((1, D), F32))
        else:
            wn2e1.append(jnp.concatenate(
                [pa1["Wn2e"][:128], pb1["Wn2e"][:128]], axis=1))
            wt1.append(jnp.concatenate(
                [pa1["Wn2e"][128:129], pb1["Wn2e"][128:129]], axis=1))
            wnode1.append(jnp.concatenate(
                [pa1["Wnode"][:128], pb1["Wnode"][:128]], axis=1))
            wtnode1.append(jnp.concatenate(
                [pa1["Wnode"][128:129], pb1["Wnode"][128:129]], axis=1))
        bnode1.append(jnp.concatenate([pa1["bnode"], pb1["bnode"]]))
        we2n1.append(_blkdiag(pa1["We2n"], pb1["We2n"]))
        wgp2.append(_blkdiag(pa2["Wgp"], pb2["Wgp"]))
        bgp2.append(jnp.concatenate([pa2["bgp"], pb2["bgp"]]))
        wn2e2.append(_blkdiag(pa2["Wn2e"], pb2["Wn2e"]))
        wnode2.append(_blkdiag(pa2["Wnode"], pb2["Wnode"]))
        bnode2.append(jnp.concatenate([pa2["bnode"], pb2["bnode"]]))
        we2n2.append(_blkdiag(pa2["We2n"], pb2["We2n"]))

    stk = lambda xs: jnp.stack(xs)
    stkb = lambda xs: jnp.stack(xs).reshape(2, 1, D)

    g1, g2, sc = _sc_kernels()

    nb = N_EDGES // EBLK
    src3 = src.reshape(nb, EBLK, 1)
    dst3 = dst.reshape(nb, EBLK, 1)
    tgrid_t = jnp.pad(feature_T[:, 0], (0, 128 * 128 - N_NODES)) \
        .reshape(128, 128).T
    z7 = jnp.zeros((2, 7, D), F32)
    wn2e1_ext = jnp.concatenate(
        [stk(wn2e1), jnp.stack(wt1).reshape(2, 1, D), z7], axis=1)
    wnode1_ext = jnp.concatenate(
        [stk(wnode1), jnp.stack(wtnode1).reshape(2, 1, D), z7], axis=1)

    msg1 = g1(feature_Node, src, dst)
    tsum = _tsum(src3, dst3, tgrid_t)
    h1c = _edge1(feature_GP, msg1, tsum.reshape(nb, EBLK, 1),
                 stk(w1gp), wn2e1_ext, stkb(b1gp))
    agg1p = sc(h1c, dst)
    agg1 = agg1p.reshape(2, NPAD, D)[:, :N_NODES].reshape(2 * N_NODES, D)
    deg = _deg(dst3).reshape(128 * 128, 1)[:N_NODES]
    h2c = _node_upd(agg1, deg, stk(we2n1), feature_Node, feature_T,
                    wnode1_ext, stkb(bnode1))

    msg2 = g2(h2c, src, dst)
    h1pc = _edge2(h1c, msg2, stk(wgp2), stk(wn2e2), stkb(bgp2))
    agg2p = sc(h1pc, dst)
    agg2 = agg2p.reshape(2, NPAD, D)[:, :N_NODES]

    wha = jnp.zeros((D, 3), F32)
    wha = wha.at[0:64, 0:2].set(p["linear"]["W"])
    wha = wha.at[64:128, 2:3].set(p["Elinear"]["W"])
    bha = jnp.concatenate([p["linear"]["b"], p["Elinear"]["b"]]).reshape(1, 3)
    whb = jnp.zeros((D, 2), F32)
    whb = whb.at[0:64, 0:1].set(p["Blinear"]["W"])
    whb = whb.at[64:128, 1:2].set(p["siglinear"]["W"])
    bhb = jnp.concatenate([p["Blinear"]["b"], p["siglinear"]["b"]]).reshape(1, 2)

    tan6 = feature_tan.reshape(N_NODES, 6)
    out, sig = _final(agg2[0], agg2[1], deg, h2c[:N_NODES], h2c[N_NODES:],
                      we2n2[0], we2n2[1], wnode2[0], wnode2[1],
                      bnode2[0].reshape(1, D), bnode2[1].reshape(1, D),
                      wha, bha, whb, bhb, tan6, feature_T)
    return out, sig
